# baseline (speedup 1.0000x reference)
import functools

import jax
import jax.numpy as jnp
from jax import lax
from jax.experimental import pallas as pl
from jax.experimental.pallas import tpu as pltpu
from jax.experimental.pallas import tpu_sc as plsc

NSEG = 16
DFEAT = 128
BLOCK = 6400
NWORK = 32
SC_LANES = 16


def _sc_counts_kernel(seg_hbm, out_hbm, ids_v, acc_v):
    wid = lax.axis_index("s") * 2 + lax.axis_index("c")
    chunk = ids_v.shape[0]
    pltpu.sync_copy(seg_hbm.at[pl.ds(wid * chunk, chunk)], ids_v)

    niter = chunk // SC_LANES
    one = jnp.ones((SC_LANES,), jnp.int32)
    init = tuple(jnp.zeros((SC_LANES,), jnp.int32) for _ in range(NSEG))

    def body(i, ne_accs):
        v = ids_v[pl.ds(i * SC_LANES, SC_LANES)]
        return tuple(ne_accs[s] + jnp.minimum(v ^ s, one)
                     for s in range(NSEG))

    ne_accs = lax.fori_loop(0, niter, body, init)
    full = jnp.full((SC_LANES,), niter, jnp.int32)
    for s in range(NSEG):
        acc_v[s, :] = (full - ne_accs[s]).astype(jnp.float32)
    pltpu.sync_copy(acc_v, out_hbm.at[wid])


def _sc_counts(seg_i32):
    n = seg_i32.shape[0]
    chunk = n // NWORK
    mesh = plsc.VectorSubcoreMesh(core_axis_name="c", subcore_axis_name="s")
    return pl.kernel(
        _sc_counts_kernel,
        mesh=mesh,
        out_type=jax.ShapeDtypeStruct((NWORK, NSEG, SC_LANES), jnp.float32),
        scratch_types=[
            pltpu.VMEM((chunk,), jnp.int32),
            pltpu.VMEM((NSEG, SC_LANES), jnp.float32),
        ],
    )(seg_i32)


def _stats_kernel(seg_ref, x_ref, sums_ref, sumsq_ref):
    i = pl.program_id(0)
    x = x_ref[...]
    seg = seg_ref[0, 0, :]
    onehot = (seg[:, None] == jax.lax.broadcasted_iota(
        jnp.int32, (x.shape[0], NSEG), 1)).astype(jnp.bfloat16)
    xb = x.astype(jnp.bfloat16)
    dim = (((0,), (0,)), ((), ()))
    psums = jax.lax.dot_general(onehot, xb, dim,
                                preferred_element_type=jnp.float32)
    psumsq = jax.lax.dot_general(onehot, xb * xb, dim,
                                 preferred_element_type=jnp.float32)

    @pl.when(i == 0)
    def _init():
        sums_ref[...] = psums
        sumsq_ref[...] = psumsq

    @pl.when(i > 0)
    def _acc():
        sums_ref[...] += psums
        sumsq_ref[...] += psumsq


def _apply_kernel(seg_ref, x_ref, sums_ref, sumsq_ref, counts_ref,
                  w_ref, b_ref, out_ref):
    x = x_ref[...]
    seg = seg_ref[0, 0, :]
    n = jnp.maximum(jnp.sum(counts_ref[...], axis=(0, 2)), 1.0)[:, None]
    mean = sums_ref[...] / n
    var = sumsq_ref[...] / n - mean * mean
    inv = jax.lax.rsqrt(jnp.maximum(var, 0.0) + 1e-8)
    scale = inv * w_ref[...]
    shift = b_ref[...] - mean * scale
    onehot = (seg[:, None] == jax.lax.broadcasted_iota(
        jnp.int32, (x.shape[0], NSEG), 1)).astype(jnp.float32)
    rowscale = jnp.dot(onehot, scale, preferred_element_type=jnp.float32)
    rowshift = jnp.dot(onehot, shift, preferred_element_type=jnp.float32)
    out_ref[...] = x * rowscale + rowshift


@jax.jit
def _run(in_feat, segment_ids, weight, bias):
    n_rows, d = in_feat.shape
    nblk = n_rows // BLOCK
    seg_i32 = segment_ids.astype(jnp.int32)
    seg3d = seg_i32.reshape(nblk, 1, BLOCK)

    counts32 = _sc_counts(seg_i32)

    stats_shape = jax.ShapeDtypeStruct((NSEG, DFEAT), jnp.float32)
    sums, sumsq = pl.pallas_call(
        _stats_kernel,
        grid=(nblk,),
        in_specs=[
            pl.BlockSpec((1, 1, BLOCK), lambda i: (i, 0, 0)),
            pl.BlockSpec((BLOCK, d), lambda i: (i, 0)),
        ],
        out_specs=[
            pl.BlockSpec((NSEG, DFEAT), lambda i: (0, 0)),
            pl.BlockSpec((NSEG, DFEAT), lambda i: (0, 0)),
        ],
        out_shape=[stats_shape, stats_shape],
    )(seg3d, in_feat)

    out = pl.pallas_call(
        _apply_kernel,
        grid=(nblk,),
        in_specs=[
            pl.BlockSpec((1, 1, BLOCK), lambda i: (i, 0, 0)),
            pl.BlockSpec((BLOCK, d), lambda i: (i, 0)),
            pl.BlockSpec((NSEG, DFEAT), lambda i: (0, 0)),
            pl.BlockSpec((NSEG, DFEAT), lambda i: (0, 0)),
            pl.BlockSpec((NWORK, NSEG, SC_LANES), lambda i: (0, 0, 0)),
            pl.BlockSpec((1, DFEAT), lambda i: (0, 0)),
            pl.BlockSpec((1, DFEAT), lambda i: (0, 0)),
        ],
        out_specs=pl.BlockSpec((BLOCK, d), lambda i: (i, 0)),
        out_shape=jax.ShapeDtypeStruct((n_rows, d), jnp.float32),
    )(seg3d, in_feat, sums, sumsq, counts32, weight, bias)
    return out


def kernel(in_feat, segment_ids, weight, bias):
    return _run(in_feat, segment_ids, weight, bias)


# speedup vs baseline: 1.1290x; 1.1290x over previous
import functools

import jax
import jax.numpy as jnp
from jax import lax
from jax.experimental import pallas as pl
from jax.experimental.pallas import tpu as pltpu
from jax.experimental.pallas import tpu_sc as plsc

NSEG = 16
DFEAT = 128
BLOCK = 6400
NWORK = 32
SC_LANES = 16


def _sc_counts_kernel(seg_hbm, out_hbm, ids_v, acc_v):
    wid = lax.axis_index("s") * 2 + lax.axis_index("c")
    chunk = ids_v.shape[0]
    pltpu.sync_copy(seg_hbm.at[pl.ds(wid * chunk, chunk)], ids_v)

    niter = chunk // SC_LANES
    one = jnp.ones((SC_LANES,), jnp.int32)
    init = tuple(jnp.zeros((SC_LANES,), jnp.int32) for _ in range(NSEG))

    def body(i, ne_accs):
        v = ids_v[pl.ds(i * SC_LANES, SC_LANES)]
        return tuple(ne_accs[s] + jnp.minimum(v ^ s, one)
                     for s in range(NSEG))

    ne_accs = lax.fori_loop(0, niter, body, init)
    full = jnp.full((SC_LANES,), niter, jnp.int32)
    for s in range(NSEG):
        acc_v[s, :] = (full - ne_accs[s]).astype(jnp.float32)
    pltpu.sync_copy(acc_v, out_hbm.at[wid])


def _sc_counts(seg_i32):
    n = seg_i32.shape[0]
    chunk = n // NWORK
    mesh = plsc.VectorSubcoreMesh(core_axis_name="c", subcore_axis_name="s")
    return pl.kernel(
        _sc_counts_kernel,
        mesh=mesh,
        out_type=jax.ShapeDtypeStruct((NWORK, NSEG, SC_LANES), jnp.float32),
        scratch_types=[
            pltpu.VMEM((chunk,), jnp.int32),
            pltpu.VMEM((NSEG, SC_LANES), jnp.float32),
        ],
    )(seg_i32)


def _stats_kernel(seg_ref, x_ref, sums_ref, sumsq_ref):
    i = pl.program_id(0)
    x = x_ref[...]
    seg = seg_ref[0, 0, :]
    onehot_t = (jax.lax.broadcasted_iota(
        jnp.int32, (NSEG, x.shape[0]), 0) == seg[None, :]).astype(jnp.bfloat16)
    xb = x.astype(jnp.bfloat16)
    dim = (((1,), (0,)), ((), ()))
    psums = jax.lax.dot_general(onehot_t, xb, dim,
                                preferred_element_type=jnp.float32)
    psumsq = jax.lax.dot_general(onehot_t, xb * xb, dim,
                                 preferred_element_type=jnp.float32)

    @pl.when(i == 0)
    def _init():
        sums_ref[...] = psums
        sumsq_ref[...] = psumsq

    @pl.when(i > 0)
    def _acc():
        sums_ref[...] += psums
        sumsq_ref[...] += psumsq


def _apply_kernel(seg_ref, x_ref, sums_ref, sumsq_ref, counts_ref,
                  w_ref, b_ref, out_ref):
    x = x_ref[...]
    seg = seg_ref[0, 0, :]
    n = jnp.maximum(jnp.sum(counts_ref[...], axis=(0, 2)), 1.0)[:, None]
    mean = sums_ref[...] / n
    var = sumsq_ref[...] / n - mean * mean
    inv = jax.lax.rsqrt(jnp.maximum(var, 0.0) + 1e-8)
    scale = inv * w_ref[...]
    shift = b_ref[...] - mean * scale
    onehot_t = (jax.lax.broadcasted_iota(
        jnp.int32, (NSEG, x.shape[0]), 0) == seg[None, :]).astype(jnp.float32)
    dim = (((0,), (0,)), ((), ()))
    rowscale = jax.lax.dot_general(onehot_t, scale, dim,
                                   preferred_element_type=jnp.float32)
    rowshift = jax.lax.dot_general(onehot_t, shift, dim,
                                   preferred_element_type=jnp.float32)
    out_ref[...] = x * rowscale + rowshift


@jax.jit
def _run(in_feat, segment_ids, weight, bias):
    n_rows, d = in_feat.shape
    nblk = n_rows // BLOCK
    seg_i32 = segment_ids.astype(jnp.int32)
    seg3d = seg_i32.reshape(nblk, 1, BLOCK)

    counts32 = _sc_counts(seg_i32)

    stats_shape = jax.ShapeDtypeStruct((NSEG, DFEAT), jnp.float32)
    sums, sumsq = pl.pallas_call(
        _stats_kernel,
        grid=(nblk,),
        in_specs=[
            pl.BlockSpec((1, 1, BLOCK), lambda i: (i, 0, 0)),
            pl.BlockSpec((BLOCK, d), lambda i: (i, 0)),
        ],
        out_specs=[
            pl.BlockSpec((NSEG, DFEAT), lambda i: (0, 0)),
            pl.BlockSpec((NSEG, DFEAT), lambda i: (0, 0)),
        ],
        out_shape=[stats_shape, stats_shape],
    )(seg3d, in_feat)

    out = pl.pallas_call(
        _apply_kernel,
        grid=(nblk,),
        in_specs=[
            pl.BlockSpec((1, 1, BLOCK), lambda i: (i, 0, 0)),
            pl.BlockSpec((BLOCK, d), lambda i: (i, 0)),
            pl.BlockSpec((NSEG, DFEAT), lambda i: (0, 0)),
            pl.BlockSpec((NSEG, DFEAT), lambda i: (0, 0)),
            pl.BlockSpec((NWORK, NSEG, SC_LANES), lambda i: (0, 0, 0)),
            pl.BlockSpec((1, DFEAT), lambda i: (0, 0)),
            pl.BlockSpec((1, DFEAT), lambda i: (0, 0)),
        ],
        out_specs=pl.BlockSpec((BLOCK, d), lambda i: (i, 0)),
        out_shape=jax.ShapeDtypeStruct((n_rows, d), jnp.float32),
    )(seg3d, in_feat, sums, sumsq, counts32, weight, bias)
    return out


def kernel(in_feat, segment_ids, weight, bias):
    return _run(in_feat, segment_ids, weight, bias)


# speedup vs baseline: 1.2810x; 1.1347x over previous
import functools

import jax
import jax.numpy as jnp
from jax import lax
from jax.experimental import pallas as pl
from jax.experimental.pallas import tpu as pltpu
from jax.experimental.pallas import tpu_sc as plsc

NSEG = 16
DFEAT = 128
BLOCK = 12800
NWORK = 32
SC_LANES = 16


def _sc_counts_kernel(seg_hbm, out_hbm, ids_v, acc_v):
    wid = lax.axis_index("s") * 2 + lax.axis_index("c")
    chunk = ids_v.shape[0]
    pltpu.sync_copy(seg_hbm.at[pl.ds(wid * chunk, chunk)], ids_v)

    niter = chunk // SC_LANES
    one = jnp.ones((SC_LANES,), jnp.int32)
    init = tuple(jnp.zeros((SC_LANES,), jnp.int32) for _ in range(NSEG))

    def body(i, ne_accs):
        v = ids_v[pl.ds(i * SC_LANES, SC_LANES)]
        return tuple(ne_accs[s] + jnp.minimum(v ^ s, one)
                     for s in range(NSEG))

    ne_accs = lax.fori_loop(0, niter, body, init)
    full = jnp.full((SC_LANES,), niter, jnp.int32)
    for s in range(NSEG):
        acc_v[s, :] = (full - ne_accs[s]).astype(jnp.float32)
    pltpu.sync_copy(acc_v, out_hbm.at[wid])


def _sc_counts(seg_i32):
    n = seg_i32.shape[0]
    chunk = n // NWORK
    mesh = plsc.VectorSubcoreMesh(core_axis_name="c", subcore_axis_name="s")
    return pl.kernel(
        _sc_counts_kernel,
        mesh=mesh,
        out_type=jax.ShapeDtypeStruct((NWORK, NSEG, SC_LANES), jnp.float32),
        scratch_types=[
            pltpu.VMEM((chunk,), jnp.int32),
            pltpu.VMEM((NSEG, SC_LANES), jnp.float32),
        ],
    )(seg_i32)


def _stats_kernel(seg_ref, x_ref, sums_ref, sumsq_ref):
    i = pl.program_id(0)
    x = x_ref[...]
    seg = seg_ref[0, 0, :]
    onehot_t = (jax.lax.broadcasted_iota(
        jnp.int32, (NSEG, x.shape[0]), 0) == seg[None, :]).astype(jnp.bfloat16)
    xb = x.astype(jnp.bfloat16)
    dim = (((1,), (0,)), ((), ()))
    psums = jax.lax.dot_general(onehot_t, xb, dim,
                                preferred_element_type=jnp.float32)
    psumsq = jax.lax.dot_general(onehot_t, xb * xb, dim,
                                 preferred_element_type=jnp.float32)

    @pl.when(i == 0)
    def _init():
        sums_ref[...] = psums
        sumsq_ref[...] = psumsq

    @pl.when(i > 0)
    def _acc():
        sums_ref[...] += psums
        sumsq_ref[...] += psumsq


def _apply_kernel(seg_ref, x_ref, sums_ref, sumsq_ref, counts_ref,
                  w_ref, b_ref, out_ref):
    x = x_ref[...]
    seg = seg_ref[0, 0, :]
    n = jnp.maximum(jnp.sum(counts_ref[...], axis=(0, 2)), 1.0)[:, None]
    mean = sums_ref[...] / n
    var = sumsq_ref[...] / n - mean * mean
    inv = jax.lax.rsqrt(jnp.maximum(var, 0.0) + 1e-8)
    scale = inv * w_ref[...]
    shift = b_ref[...] - mean * scale
    onehot_t = (jax.lax.broadcasted_iota(
        jnp.int32, (NSEG, x.shape[0]), 0) == seg[None, :]).astype(jnp.float32)
    dim = (((0,), (0,)), ((), ()))
    rowscale = jax.lax.dot_general(onehot_t, scale, dim,
                                   preferred_element_type=jnp.float32)
    rowshift = jax.lax.dot_general(onehot_t, shift, dim,
                                   preferred_element_type=jnp.float32)
    out_ref[...] = x * rowscale + rowshift


@jax.jit
def _run(in_feat, segment_ids, weight, bias):
    n_rows, d = in_feat.shape
    nblk = n_rows // BLOCK
    seg_i32 = segment_ids.astype(jnp.int32)
    seg3d = seg_i32.reshape(nblk, 1, BLOCK)

    counts32 = _sc_counts(seg_i32)

    stats_shape = jax.ShapeDtypeStruct((NSEG, DFEAT), jnp.float32)
    sums, sumsq = pl.pallas_call(
        _stats_kernel,
        grid=(nblk,),
        in_specs=[
            pl.BlockSpec((1, 1, BLOCK), lambda i: (i, 0, 0)),
            pl.BlockSpec((BLOCK, d), lambda i: (i, 0)),
        ],
        out_specs=[
            pl.BlockSpec((NSEG, DFEAT), lambda i: (0, 0)),
            pl.BlockSpec((NSEG, DFEAT), lambda i: (0, 0)),
        ],
        out_shape=[stats_shape, stats_shape],
    )(seg3d, in_feat)

    out = pl.pallas_call(
        _apply_kernel,
        grid=(nblk,),
        in_specs=[
            pl.BlockSpec((1, 1, BLOCK), lambda i: (i, 0, 0)),
            pl.BlockSpec((BLOCK, d), lambda i: (i, 0)),
            pl.BlockSpec((NSEG, DFEAT), lambda i: (0, 0)),
            pl.BlockSpec((NSEG, DFEAT), lambda i: (0, 0)),
            pl.BlockSpec((NWORK, NSEG, SC_LANES), lambda i: (0, 0, 0)),
            pl.BlockSpec((1, DFEAT), lambda i: (0, 0)),
            pl.BlockSpec((1, DFEAT), lambda i: (0, 0)),
        ],
        out_specs=pl.BlockSpec((BLOCK, d), lambda i: (i, 0)),
        out_shape=jax.ShapeDtypeStruct((n_rows, d), jnp.float32),
    )(seg3d, in_feat, sums, sumsq, counts32, weight, bias)
    return out


def kernel(in_feat, segment_ids, weight, bias):
    return _run(in_feat, segment_ids, weight, bias)


# speedup vs baseline: 1.3043x; 1.0182x over previous
import functools

import jax
import jax.numpy as jnp
from jax import lax
from jax.experimental import pallas as pl
from jax.experimental.pallas import tpu as pltpu
from jax.experimental.pallas import tpu_sc as plsc

NSEG = 16
DFEAT = 128
BLOCK = 20000
NWORK = 32
SC_LANES = 16


def _sc_counts_kernel(seg_hbm, out_hbm, ids_v, acc_v):
    wid = lax.axis_index("s") * 2 + lax.axis_index("c")
    chunk = ids_v.shape[0]
    pltpu.sync_copy(seg_hbm.at[pl.ds(wid * chunk, chunk)], ids_v)

    niter = chunk // SC_LANES
    one = jnp.ones((SC_LANES,), jnp.int32)
    init = tuple(jnp.zeros((SC_LANES,), jnp.int32) for _ in range(NSEG))

    def body(i, ne_accs):
        v = ids_v[pl.ds(i * SC_LANES, SC_LANES)]
        return tuple(ne_accs[s] + jnp.minimum(v ^ s, one)
                     for s in range(NSEG))

    ne_accs = lax.fori_loop(0, niter, body, init)
    full = jnp.full((SC_LANES,), niter, jnp.int32)
    for s in range(NSEG):
        acc_v[s, :] = (full - ne_accs[s]).astype(jnp.float32)
    pltpu.sync_copy(acc_v, out_hbm.at[wid])


def _sc_counts(seg_i32):
    n = seg_i32.shape[0]
    chunk = n // NWORK
    mesh = plsc.VectorSubcoreMesh(core_axis_name="c", subcore_axis_name="s")
    return pl.kernel(
        _sc_counts_kernel,
        mesh=mesh,
        out_type=jax.ShapeDtypeStruct((NWORK, NSEG, SC_LANES), jnp.float32),
        scratch_types=[
            pltpu.VMEM((chunk,), jnp.int32),
            pltpu.VMEM((NSEG, SC_LANES), jnp.float32),
        ],
    )(seg_i32)


def _stats_kernel(seg_ref, x_ref, sums_ref, sumsq_ref):
    i = pl.program_id(0)
    x = x_ref[...]
    seg = seg_ref[0, 0, :]
    onehot_t = (jax.lax.broadcasted_iota(
        jnp.int32, (NSEG, x.shape[0]), 0) == seg[None, :]).astype(jnp.bfloat16)
    xb = x.astype(jnp.bfloat16)
    dim = (((1,), (0,)), ((), ()))
    psums = jax.lax.dot_general(onehot_t, xb, dim,
                                preferred_element_type=jnp.float32)
    psumsq = jax.lax.dot_general(onehot_t, xb * xb, dim,
                                 preferred_element_type=jnp.float32)

    @pl.when(i == 0)
    def _init():
        sums_ref[...] = psums
        sumsq_ref[...] = psumsq

    @pl.when(i > 0)
    def _acc():
        sums_ref[...] += psums
        sumsq_ref[...] += psumsq


def _apply_kernel(seg_ref, x_ref, sums_ref, sumsq_ref, counts_ref,
                  w_ref, b_ref, out_ref):
    x = x_ref[...]
    seg = seg_ref[0, 0, :]
    n = jnp.maximum(jnp.sum(counts_ref[...], axis=(0, 2)), 1.0)[:, None]
    mean = sums_ref[...] / n
    var = sumsq_ref[...] / n - mean * mean
    inv = jax.lax.rsqrt(jnp.maximum(var, 0.0) + 1e-8)
    scale = inv * w_ref[...]
    shift = b_ref[...] - mean * scale
    onehot_t = (jax.lax.broadcasted_iota(
        jnp.int32, (NSEG, x.shape[0]), 0) == seg[None, :]).astype(jnp.float32)
    dim = (((0,), (0,)), ((), ()))
    rowscale = jax.lax.dot_general(onehot_t, scale, dim,
                                   preferred_element_type=jnp.float32)
    rowshift = jax.lax.dot_general(onehot_t, shift, dim,
                                   preferred_element_type=jnp.float32)
    out_ref[...] = x * rowscale + rowshift


@jax.jit
def _run(in_feat, segment_ids, weight, bias):
    n_rows, d = in_feat.shape
    nblk = n_rows // BLOCK
    seg_i32 = segment_ids.astype(jnp.int32)
    seg3d = seg_i32.reshape(nblk, 1, BLOCK)

    counts32 = _sc_counts(seg_i32)

    stats_shape = jax.ShapeDtypeStruct((NSEG, DFEAT), jnp.float32)
    sums, sumsq = pl.pallas_call(
        _stats_kernel,
        grid=(nblk,),
        in_specs=[
            pl.BlockSpec((1, 1, BLOCK), lambda i: (i, 0, 0)),
            pl.BlockSpec((BLOCK, d), lambda i: (i, 0)),
        ],
        out_specs=[
            pl.BlockSpec((NSEG, DFEAT), lambda i: (0, 0)),
            pl.BlockSpec((NSEG, DFEAT), lambda i: (0, 0)),
        ],
        out_shape=[stats_shape, stats_shape],
    )(seg3d, in_feat)

    out = pl.pallas_call(
        _apply_kernel,
        grid=(nblk,),
        in_specs=[
            pl.BlockSpec((1, 1, BLOCK), lambda i: (i, 0, 0)),
            pl.BlockSpec((BLOCK, d), lambda i: (i, 0)),
            pl.BlockSpec((NSEG, DFEAT), lambda i: (0, 0)),
            pl.BlockSpec((NSEG, DFEAT), lambda i: (0, 0)),
            pl.BlockSpec((NWORK, NSEG, SC_LANES), lambda i: (0, 0, 0)),
            pl.BlockSpec((1, DFEAT), lambda i: (0, 0)),
            pl.BlockSpec((1, DFEAT), lambda i: (0, 0)),
        ],
        out_specs=pl.BlockSpec((BLOCK, d), lambda i: (i, 0)),
        out_shape=jax.ShapeDtypeStruct((n_rows, d), jnp.float32),
    )(seg3d, in_feat, sums, sumsq, counts32, weight, bias)
    return out


def kernel(in_feat, segment_ids, weight, bias):
    return _run(in_feat, segment_ids, weight, bias)
